# baseline (speedup 1.0000x reference)
import functools

import jax
import jax.numpy as jnp
from jax import lax
from jax.experimental import pallas as pl
from jax.experimental.pallas import tpu as pltpu
from jax.experimental.pallas import tpu_sc as plsc

K = 16
EPSILON = 1e-4

ROWS = 64
COLS = 4096
LANES = 16
NWORKERS = 32
ROWS_PER_W = ROWS // NWORKERS
NVREG = COLS // LANES
STEP = 4
UNROLL = 2

_GATHER_DNUMS = lax.GatherDimensionNumbers(
    offset_dims=(), collapsed_slice_dims=(0,), start_index_map=(0,))


def _shuffle(v, idx):
    return lax.gather(v, idx[:, None], _GATHER_DNUMS, (1,),
                      mode=lax.GatherScatterMode.PROMISE_IN_BOUNDS)


def _lane_reduce(v, op):
    idx0 = lax.iota(jnp.int32, LANES)
    for k in (1, 2, 4, 8):
        v = op(v, _shuffle(v, jnp.bitwise_xor(idx0, k)))
    return v


def _sl(j):
    return pl.ds(j * LANES, LANES)


def _rows_body(p_ref, kh_ref):
    ninf = jnp.full((LANES,), -jnp.inf, jnp.float32)
    zero = jnp.zeros((LANES,), jnp.float32)

    def _tree_sum(vs):
        while len(vs) > 1:
            vs = [a + b for a, b in zip(vs[::2], vs[1::2])]
        return vs[0]

    def addmax_body(j, carry):
        m0s, m1s = list(carry[0]), list(carry[1])
        for u in range(STEP):
            s = _sl(j + u)
            x0 = p_ref[0, s] + kh_ref[0, s]
            x1 = p_ref[1, s] + kh_ref[1, s]
            p_ref[0, s] = x0
            p_ref[1, s] = x1
            m0s[u] = jnp.maximum(m0s[u], x0)
            m1s[u] = jnp.maximum(m1s[u], x1)
        return tuple(m0s), tuple(m1s)

    m0s, m1s = plsc.parallel_loop(
        0, NVREG, step=STEP, unroll=UNROLL,
        carry=((ninf,) * STEP, (ninf,) * STEP))(addmax_body)
    m0 = _lane_reduce(jnp.maximum(jnp.maximum(m0s[0], m0s[1]),
                                  jnp.maximum(m0s[2], m0s[3])), jnp.maximum)
    m1 = _lane_reduce(jnp.maximum(jnp.maximum(m1s[0], m1s[1]),
                                  jnp.maximum(m1s[2], m1s[3])), jnp.maximum)

    def exp_body(j, carry):
        a0s, a1s = list(carry[0]), list(carry[1])
        for u in range(STEP):
            s = _sl(j + u)
            e0 = jnp.exp(p_ref[0, s] - m0)
            e1 = jnp.exp(p_ref[1, s] - m1)
            p_ref[0, s] = e0
            p_ref[1, s] = e1
            a0s[u] = a0s[u] + e0
            a1s[u] = a1s[u] + e1
        return tuple(a0s), tuple(a1s)

    a0s, a1s = plsc.parallel_loop(
        0, NVREG, step=STEP, unroll=UNROLL,
        carry=((zero,) * STEP, (zero,) * STEP))(exp_body)
    a0, a1 = _tree_sum(list(a0s)), _tree_sum(list(a1s))

    rz0 = 1.0 / _lane_reduce(a0, jnp.add)
    rz1 = 1.0 / _lane_reduce(a1, jnp.add)

    def peel_body(j, carry):
        a0s, a1s = list(carry[0]), list(carry[1])
        for u in range(STEP):
            s = _sl(j + u)
            pj0 = p_ref[0, s] * rz0
            pj1 = p_ref[1, s] * rz1
            kh_ref[0, s] = pj0
            kh_ref[1, s] = pj1
            q0 = pj0 * jnp.maximum(1.0 - pj0, EPSILON)
            q1 = pj1 * jnp.maximum(1.0 - pj1, EPSILON)
            p_ref[0, s] = q0
            p_ref[1, s] = q1
            a0s[u] = a0s[u] + q0
            a1s[u] = a1s[u] + q1
        return tuple(a0s), tuple(a1s)

    a0s, a1s = plsc.parallel_loop(
        0, NVREG, step=STEP, unroll=UNROLL,
        carry=((zero,) * STEP, (zero,) * STEP))(peel_body)
    a0, a1 = _tree_sum(list(a0s)), _tree_sum(list(a1s))

    def iter_body(_, carry):
        a0, a1 = carry
        rz0 = 1.0 / _lane_reduce(a0, jnp.add)
        rz1 = 1.0 / _lane_reduce(a1, jnp.add)

        def body(j, carry):
            a0s, a1s = list(carry[0]), list(carry[1])
            for u in range(STEP):
                s = _sl(j + u)
                pj0 = p_ref[0, s] * rz0
                pj1 = p_ref[1, s] * rz1
                kh_ref[0, s] = kh_ref[0, s] + pj0
                kh_ref[1, s] = kh_ref[1, s] + pj1
                q0 = pj0 * jnp.maximum(1.0 - pj0, EPSILON)
                q1 = pj1 * jnp.maximum(1.0 - pj1, EPSILON)
                p_ref[0, s] = q0
                p_ref[1, s] = q1
                a0s[u] = a0s[u] + q0
                a1s[u] = a1s[u] + q1
            return tuple(a0s), tuple(a1s)

        a0s, a1s = plsc.parallel_loop(
            0, NVREG, step=STEP, unroll=UNROLL,
            carry=((zero,) * STEP, (zero,) * STEP))(body)
        return _tree_sum(list(a0s)), _tree_sum(list(a1s))

    a0, a1 = lax.fori_loop(0, K - 2, iter_body, (a0, a1))

    rz0 = 1.0 / _lane_reduce(a0, jnp.add)
    rz1 = 1.0 / _lane_reduce(a1, jnp.add)

    def fin_body(j, carry):
        for u in range(STEP):
            s = _sl(j + u)
            kh_ref[0, s] = kh_ref[0, s] + p_ref[0, s] * rz0
            kh_ref[1, s] = kh_ref[1, s] + p_ref[1, s] * rz1
        return carry

    plsc.parallel_loop(0, NVREG, step=STEP, unroll=UNROLL,
                       carry=jnp.int32(0))(fin_body)


def _sc_call(scores, gumbel):
    mesh = plsc.VectorSubcoreMesh(core_axis_name="c", subcore_axis_name="s")

    @functools.partial(
        pl.kernel,
        mesh=mesh,
        out_type=jax.ShapeDtypeStruct((ROWS, COLS), jnp.float32),
        scratch_types=[
            pltpu.VMEM((ROWS_PER_W, COLS), jnp.float32),
            pltpu.VMEM((ROWS_PER_W, COLS), jnp.float32),
        ],
    )
    def k(scores_hbm, gum_hbm, out_hbm, p_v, kh_v):
        wid = lax.axis_index("s") * 2 + lax.axis_index("c")
        base = wid * ROWS_PER_W
        pltpu.sync_copy(scores_hbm.at[pl.ds(base, ROWS_PER_W)], p_v)
        pltpu.sync_copy(gum_hbm.at[pl.ds(base, ROWS_PER_W)], kh_v)
        _rows_body(p_v, kh_v)
        pltpu.sync_copy(kh_v, out_hbm.at[pl.ds(base, ROWS_PER_W)])

    return k(scores, gumbel)


def kernel(scores):
    gkey = jax.random.fold_in(jax.random.key(0), 1)
    g = jax.random.gumbel(gkey, scores.shape, dtype=scores.dtype)
    return _sc_call(scores, g)


# speedup vs baseline: 1.2731x; 1.2731x over previous
import functools

import jax
import jax.numpy as jnp
from jax import lax
from jax.experimental import pallas as pl
from jax.experimental.pallas import tpu as pltpu
from jax.experimental.pallas import tpu_sc as plsc

K = 16
EPSILON = 1e-4

ROWS = 64
COLS = 4096
LANES = 16
NWORKERS = 32
ROWS_PER_W = ROWS // NWORKERS
NVREG = COLS // LANES
UNROLL = 8

_GATHER_DNUMS = lax.GatherDimensionNumbers(
    offset_dims=(), collapsed_slice_dims=(0,), start_index_map=(0,))


def _shuffle(v, idx):
    return lax.gather(v, idx[:, None], _GATHER_DNUMS, (1,),
                      mode=lax.GatherScatterMode.PROMISE_IN_BOUNDS)


def _lane_reduce(v, op):
    idx0 = lax.iota(jnp.int32, LANES)
    for k in (1, 2, 4, 8):
        v = op(v, _shuffle(v, jnp.bitwise_xor(idx0, k)))
    return v


def _sl(j):
    return pl.ds(j * LANES, LANES)


def _rows_body(p_ref, kh_ref):
    ninf = jnp.full((LANES,), -jnp.inf, jnp.float32)
    zero = jnp.zeros((LANES,), jnp.float32)

    def addmax_body(j, carry):
        m0, m1 = carry
        s = _sl(j)
        x0 = p_ref[0, s] + kh_ref[0, s]
        x1 = p_ref[1, s] + kh_ref[1, s]
        p_ref[0, s] = x0
        p_ref[1, s] = x1
        return jnp.maximum(m0, x0), jnp.maximum(m1, x1)

    m0, m1 = plsc.parallel_loop(0, NVREG, unroll=UNROLL,
                                carry=(ninf, ninf))(addmax_body)
    m0 = _lane_reduce(m0, jnp.maximum)
    m1 = _lane_reduce(m1, jnp.maximum)

    def exp_body(j, carry):
        a0, a1 = carry
        s = _sl(j)
        e0 = jnp.exp(p_ref[0, s] - m0)
        e1 = jnp.exp(p_ref[1, s] - m1)
        p_ref[0, s] = e0
        p_ref[1, s] = e1
        return a0 + e0, a1 + e1

    a0, a1 = plsc.parallel_loop(0, NVREG, unroll=UNROLL,
                                carry=(zero, zero))(exp_body)

    rz0 = 1.0 / _lane_reduce(a0, jnp.add)
    rz1 = 1.0 / _lane_reduce(a1, jnp.add)

    def peel_body(j, carry):
        a0, a1 = carry
        s = _sl(j)
        pj0 = p_ref[0, s] * rz0
        pj1 = p_ref[1, s] * rz1
        kh_ref[0, s] = pj0
        kh_ref[1, s] = pj1
        q0 = pj0 * jnp.maximum(1.0 - pj0, EPSILON)
        q1 = pj1 * jnp.maximum(1.0 - pj1, EPSILON)
        p_ref[0, s] = q0
        p_ref[1, s] = q1
        return a0 + q0, a1 + q1

    a0, a1 = plsc.parallel_loop(0, NVREG, unroll=UNROLL,
                                carry=(zero, zero))(peel_body)

    def iter_body(_, carry):
        a0, a1 = carry
        rz0 = 1.0 / _lane_reduce(a0, jnp.add)
        rz1 = 1.0 / _lane_reduce(a1, jnp.add)

        def body(j, carry):
            a0, a1 = carry
            s = _sl(j)
            pj0 = p_ref[0, s] * rz0
            pj1 = p_ref[1, s] * rz1
            kh_ref[0, s] = kh_ref[0, s] + pj0
            kh_ref[1, s] = kh_ref[1, s] + pj1
            q0 = pj0 * jnp.maximum(1.0 - pj0, EPSILON)
            q1 = pj1 * jnp.maximum(1.0 - pj1, EPSILON)
            p_ref[0, s] = q0
            p_ref[1, s] = q1
            return a0 + q0, a1 + q1

        return plsc.parallel_loop(0, NVREG, unroll=UNROLL,
                                  carry=(zero, zero))(body)

    a0, a1 = lax.fori_loop(0, K - 2, iter_body, (a0, a1))

    rz0 = 1.0 / _lane_reduce(a0, jnp.add)
    rz1 = 1.0 / _lane_reduce(a1, jnp.add)

    def fin_body(j, carry):
        s = _sl(j)
        kh_ref[0, s] = kh_ref[0, s] + p_ref[0, s] * rz0
        kh_ref[1, s] = kh_ref[1, s] + p_ref[1, s] * rz1
        return carry

    plsc.parallel_loop(0, NVREG, unroll=UNROLL, carry=jnp.int32(0))(fin_body)


def _sc_call(scores, gumbel):
    mesh = plsc.VectorSubcoreMesh(core_axis_name="c", subcore_axis_name="s")

    @functools.partial(
        pl.kernel,
        mesh=mesh,
        out_type=jax.ShapeDtypeStruct((ROWS, COLS), jnp.float32),
        scratch_types=[
            pltpu.VMEM((ROWS_PER_W, COLS), jnp.float32),
            pltpu.VMEM((ROWS_PER_W, COLS), jnp.float32),
        ],
    )
    def k(scores_hbm, gum_hbm, out_hbm, p_v, kh_v):
        wid = lax.axis_index("s") * 2 + lax.axis_index("c")
        base = wid * ROWS_PER_W
        pltpu.sync_copy(scores_hbm.at[pl.ds(base, ROWS_PER_W)], p_v)
        pltpu.sync_copy(gum_hbm.at[pl.ds(base, ROWS_PER_W)], kh_v)
        _rows_body(p_v, kh_v)
        pltpu.sync_copy(kh_v, out_hbm.at[pl.ds(base, ROWS_PER_W)])

    return k(scores, gumbel)


def kernel(scores):
    gkey = jax.random.fold_in(jax.random.key(0), 1)
    g = jax.random.gumbel(gkey, scores.shape, dtype=scores.dtype)
    return _sc_call(scores, g)


# speedup vs baseline: 3.1417x; 2.4676x over previous
import jax
import jax.numpy as jnp
from jax.experimental import pallas as pl

K = 16
EPSILON = 1e-4
ROWS = 64
COLS = 4096
BLOCK = 8


def _relax_body(scores_ref, gum_ref, out_ref):
    x = scores_ref[...] + gum_ref[...]
    m = jnp.max(x, axis=1, keepdims=True)
    e = jnp.exp(x - m)
    p = e * (1.0 / jnp.sum(e, axis=1, keepdims=True))
    kh = p
    for _ in range(K - 1):
        q = p * jnp.maximum(1.0 - p, EPSILON)
        p = q * (1.0 / jnp.sum(q, axis=1, keepdims=True))
        kh = kh + p
    out_ref[...] = kh


def kernel(scores):
    gkey = jax.random.fold_in(jax.random.key(0), 1)
    g = jax.random.gumbel(gkey, scores.shape, dtype=scores.dtype)
    return pl.pallas_call(
        _relax_body,
        grid=(ROWS // BLOCK,),
        in_specs=[
            pl.BlockSpec((BLOCK, COLS), lambda i: (i, 0)),
            pl.BlockSpec((BLOCK, COLS), lambda i: (i, 0)),
        ],
        out_specs=pl.BlockSpec((BLOCK, COLS), lambda i: (i, 0)),
        out_shape=jax.ShapeDtypeStruct((ROWS, COLS), jnp.float32),
    )(scores, g)
